# Initial kernel scaffold; baseline (speedup 1.0000x reference)
import jax
import jax.numpy as jnp
from jax.experimental import pallas as pl


def kernel(features, edge_index, W1, att_src1, att_dst1, W2):
    raise NotImplementedError("write your pallas kernel here")


# speedup vs baseline: 14.1911x; 14.1911x over previous
import functools

import jax
import jax.numpy as jnp
from jax import lax
from jax.experimental import pallas as pl
from jax.experimental.pallas import tpu as pltpu
from jax.experimental.pallas import tpu_sc as plsc

N = 10000
E = 320000
IN_DIM, HID, OUT = 128, 512, 30

NPAD = 10240
EP = 327680
EPW = EP // 32
CH = 128
NCH = EPW // CH
D1 = IN_DIM
D2 = 32
RPT = NPAD // 16
BN = 1024

_sc_mesh = plsc.VectorSubcoreMesh(core_axis_name="c", subcore_axis_name="s")


def _tc1_body(x_ref, w1_ref, att2_ref, o_ref):
    av = jnp.dot(w1_ref[...], att2_ref[...], preferred_element_type=jnp.float32)
    o_ref[...] = jnp.dot(x_ref[...], av, preferred_element_type=jnp.float32)


def _tc1(features, W1, att2):
    return pl.pallas_call(
        _tc1_body,
        out_shape=jax.ShapeDtypeStruct((N, 2), jnp.float32),
    )(features, W1, att2)


def _elu(x):
    return jnp.where(x > 0, x, jnp.exp(x) - 1.0)


def _tc2_body(u1_ref, sp_ref, w1_ref, w2_ref, o_ref):
    s = jnp.sum(sp_ref[...], axis=0)
    inv = 1.0 / (s + 1e-16)
    g = (u1_ref[0] + u1_ref[1]) * inv[:, None]
    h1 = _elu(jnp.dot(g, w1_ref[...], preferred_element_type=jnp.float32))
    o_ref[...] = jnp.dot(h1, w2_ref[...], preferred_element_type=jnp.float32)


def _tc2(u1, sparts, W1, W2p):
    return pl.pallas_call(
        _tc2_body,
        grid=(NPAD // BN,),
        in_specs=[
            pl.BlockSpec((2, BN, D1), lambda i: (0, i, 0)),
            pl.BlockSpec((32, BN), lambda i: (0, i)),
            pl.BlockSpec((IN_DIM, HID), lambda i: (0, 0)),
            pl.BlockSpec((HID, D2), lambda i: (0, 0)),
        ],
        out_specs=pl.BlockSpec((BN, D2), lambda i: (i, 0)),
        out_shape=jax.ShapeDtypeStruct((NPAD, D2), jnp.float32),
    )(u1, sparts, W1, W2p)


def _tc3_body(u2_ref, sp_ref, w2t_ref, w1t_ref, o_ref):
    s = jnp.sum(sp_ref[...], axis=0)
    inv = 1.0 / (s + 1e-16)
    g = (u2_ref[0] + u2_ref[1]) * inv[:, None]
    h3 = _elu(jnp.dot(g, w2t_ref[...], preferred_element_type=jnp.float32))
    o_ref[...] = jnp.dot(h3, w1t_ref[...], preferred_element_type=jnp.float32)


def _tc3(u2, sparts, W2pT, W1T):
    return pl.pallas_call(
        _tc3_body,
        grid=(NPAD // BN,),
        in_specs=[
            pl.BlockSpec((2, BN, D2), lambda i: (0, i, 0)),
            pl.BlockSpec((32, BN), lambda i: (0, i)),
            pl.BlockSpec((D2, HID), lambda i: (0, 0)),
            pl.BlockSpec((HID, IN_DIM), lambda i: (0, 0)),
        ],
        out_specs=pl.BlockSpec((BN, IN_DIM), lambda i: (i, 0)),
        out_shape=jax.ShapeDtypeStruct((NPAD, IN_DIM), jnp.float32),
    )(u2, sparts, W2pT, W1T)


def _sc_a_body(feat, srcp, dstp, asrc, adst,
               u1, sparts, exv,
               asrc_t, adst_t, s_t, sidx, didx, exb, rows, uacc, sem):
    cid = lax.axis_index("c")
    sid = lax.axis_index("s")
    wid = cid * 16 + sid

    pltpu.sync_copy(asrc, asrc_t)
    pltpu.sync_copy(adst, adst_t)

    zv = jnp.zeros((16,), jnp.float32)

    @pl.loop(0, NPAD // 16)
    def _(i):
        s_t[pl.ds(i * 16, 16)] = zv

    @pl.loop(0, CH)
    def _(i):
        for j in range(D1 // 16):
            rows[i, pl.ds(j * 16, 16)] = zv

    for k in range(RPT // CH):
        pltpu.sync_copy(rows, uacc.at[pl.ds(sid * RPT + k * CH, CH)])
    plsc.subcore_barrier()

    ebase = wid * EPW

    @pl.loop(0, NCH)
    def _(c):
        base = ebase + c * CH
        pltpu.sync_copy(srcp.at[pl.ds(base, CH)], sidx)
        pltpu.sync_copy(dstp.at[pl.ds(base, CH)], didx)
        pltpu.async_copy(feat.at[sidx], rows, sem).wait()
        for j in range(CH // 16):
            sv = sidx[pl.ds(j * 16, 16)]
            dv = didx[pl.ds(j * 16, 16)]
            z = plsc.load_gather(asrc_t, [sv]) + plsc.load_gather(adst_t, [dv])
            sig = 1.0 / (1.0 + jnp.exp(-z))
            ex = jnp.exp(sig)
            exb[pl.ds(j * 16, 16)] = ex
            plsc.addupdate_scatter(s_t, [dv], ex)

        @pl.loop(0, CH)
        def _(i):
            ev = plsc.load_gather(exb, [jnp.zeros((16,), jnp.int32) + i])
            for j in range(D1 // 16):
                rows[i, pl.ds(j * 16, 16)] = rows[i, pl.ds(j * 16, 16)] * ev

        pltpu.sync_copy(rows, uacc.at[didx], add=True)
        pltpu.sync_copy(exb, exv.at[pl.ds(base, CH)])

    pltpu.sync_copy(s_t, sparts.at[wid])
    plsc.subcore_barrier()
    for k in range(RPT // CH):
        r0 = sid * RPT + k * CH
        pltpu.sync_copy(uacc.at[pl.ds(r0, CH)], u1.at[cid, pl.ds(r0, CH)])


_sc_a = functools.partial(
    pl.kernel,
    out_type=(
        jax.ShapeDtypeStruct((2, NPAD, D1), jnp.float32),
        jax.ShapeDtypeStruct((32, NPAD), jnp.float32),
        jax.ShapeDtypeStruct((EP,), jnp.float32),
    ),
    mesh=_sc_mesh,
    compiler_params=pltpu.CompilerParams(
        needs_layout_passes=False, use_tc_tiling_on_sc=False),
    scratch_types=[
        pltpu.VMEM((NPAD,), jnp.float32),
        pltpu.VMEM((NPAD,), jnp.float32),
        pltpu.VMEM((NPAD,), jnp.float32),
        pltpu.VMEM((CH,), jnp.int32),
        pltpu.VMEM((CH,), jnp.int32),
        pltpu.VMEM((CH,), jnp.float32),
        pltpu.VMEM((CH, D1), jnp.float32),
        pltpu.VMEM_SHARED((NPAD, D1), jnp.float32),
        pltpu.SemaphoreType.DMA,
    ],
)(_sc_a_body)


def _sc_b_body(h2p, srcp, dstp, exv,
               u2,
               sidx, didx, exb, rows, uacc, sem):
    cid = lax.axis_index("c")
    sid = lax.axis_index("s")
    wid = cid * 16 + sid

    zv = jnp.zeros((16,), jnp.float32)

    @pl.loop(0, CH)
    def _(i):
        for j in range(D2 // 16):
            rows[i, pl.ds(j * 16, 16)] = zv

    for k in range(RPT // CH):
        pltpu.sync_copy(rows, uacc.at[pl.ds(sid * RPT + k * CH, CH)])
    plsc.subcore_barrier()

    ebase = wid * EPW

    @pl.loop(0, NCH)
    def _(c):
        base = ebase + c * CH
        pltpu.sync_copy(srcp.at[pl.ds(base, CH)], sidx)
        pltpu.sync_copy(dstp.at[pl.ds(base, CH)], didx)
        pltpu.sync_copy(exv.at[pl.ds(base, CH)], exb)
        pltpu.async_copy(h2p.at[sidx], rows, sem).wait()

        @pl.loop(0, CH)
        def _(i):
            ev = plsc.load_gather(exb, [jnp.zeros((16,), jnp.int32) + i])
            for j in range(D2 // 16):
                rows[i, pl.ds(j * 16, 16)] = rows[i, pl.ds(j * 16, 16)] * ev

        pltpu.sync_copy(rows, uacc.at[didx], add=True)

    plsc.subcore_barrier()
    for k in range(RPT // CH):
        r0 = sid * RPT + k * CH
        pltpu.sync_copy(uacc.at[pl.ds(r0, CH)], u2.at[cid, pl.ds(r0, CH)])


_sc_b = functools.partial(
    pl.kernel,
    out_type=jax.ShapeDtypeStruct((2, NPAD, D2), jnp.float32),
    mesh=_sc_mesh,
    compiler_params=pltpu.CompilerParams(
        needs_layout_passes=False, use_tc_tiling_on_sc=False),
    scratch_types=[
        pltpu.VMEM((CH,), jnp.int32),
        pltpu.VMEM((CH,), jnp.int32),
        pltpu.VMEM((CH,), jnp.float32),
        pltpu.VMEM((CH, D2), jnp.float32),
        pltpu.VMEM_SHARED((NPAD, D2), jnp.float32),
        pltpu.SemaphoreType.DMA,
    ],
)(_sc_b_body)


def kernel(features, edge_index, W1, att_src1, att_dst1, W2):
    src = edge_index[0]
    dst = edge_index[1]
    pad_e = EP - E
    srcp = jnp.concatenate([src, jnp.zeros((pad_e,), jnp.int32)])
    dstp = jnp.concatenate([dst, jnp.full((pad_e,), N, jnp.int32)])
    att2 = jnp.stack([att_src1, att_dst1], axis=1)
    W2p = jnp.pad(W2, ((0, 0), (0, D2 - OUT)))

    a2 = _tc1(features, W1, att2)
    a2p = jnp.pad(a2, ((0, NPAD - N), (0, 0)))
    asrc = a2p[:, 0]
    adst = a2p[:, 1]

    u1, sparts, exv = _sc_a(features, srcp, dstp, asrc, adst)
    h2p = _tc2(u1, sparts, W1, W2p)
    u2 = _sc_b(h2p, srcp, dstp, exv)
    h4p = _tc3(u2, sparts, W2p.T, W1.T)
    return (h2p[:N, :OUT], h4p[:N])


# speedup vs baseline: 14.3083x; 1.0083x over previous
import functools

import jax
import jax.numpy as jnp
from jax import lax
from jax.experimental import pallas as pl
from jax.experimental.pallas import tpu as pltpu
from jax.experimental.pallas import tpu_sc as plsc

N = 10000
E = 320000
IN_DIM, HID, OUT = 128, 512, 30

NPAD = 10240
EP = 327680
EPW = EP // 32
CH = 128
NCH = EPW // CH
D1 = IN_DIM
D2 = 32
RPT = NPAD // 16
BN = 1024

_sc_mesh = plsc.VectorSubcoreMesh(core_axis_name="c", subcore_axis_name="s")


def _tc1_body(x_ref, w1_ref, att2_ref, o_ref):
    av = jnp.dot(w1_ref[...], att2_ref[...], preferred_element_type=jnp.float32)
    o_ref[...] = jnp.dot(x_ref[...], av, preferred_element_type=jnp.float32)


def _tc1(features, W1, att2):
    return pl.pallas_call(
        _tc1_body,
        out_shape=jax.ShapeDtypeStruct((N, 2), jnp.float32),
    )(features, W1, att2)


def _elu(x):
    return jnp.where(x > 0, x, jnp.exp(x) - 1.0)


def _tc2_body(u1_ref, sp_ref, w1_ref, w2_ref, o_ref):
    s = jnp.sum(sp_ref[...], axis=0)
    inv = 1.0 / (s + 1e-16)
    g = (u1_ref[0] + u1_ref[1]) * inv[:, None]
    h1 = _elu(jnp.dot(g, w1_ref[...], preferred_element_type=jnp.float32))
    o_ref[...] = jnp.dot(h1, w2_ref[...], preferred_element_type=jnp.float32)


def _tc2(u1, sparts, W1, W2p):
    return pl.pallas_call(
        _tc2_body,
        grid=(NPAD // BN,),
        in_specs=[
            pl.BlockSpec((2, BN, D1), lambda i: (0, i, 0)),
            pl.BlockSpec((32, BN), lambda i: (0, i)),
            pl.BlockSpec((IN_DIM, HID), lambda i: (0, 0)),
            pl.BlockSpec((HID, D2), lambda i: (0, 0)),
        ],
        out_specs=pl.BlockSpec((BN, D2), lambda i: (i, 0)),
        out_shape=jax.ShapeDtypeStruct((NPAD, D2), jnp.float32),
    )(u1, sparts, W1, W2p)


def _tc3_body(u2_ref, sp_ref, w2t_ref, w1t_ref, o_ref):
    s = jnp.sum(sp_ref[...], axis=0)
    inv = 1.0 / (s + 1e-16)
    g = (u2_ref[0] + u2_ref[1]) * inv[:, None]
    h3 = _elu(jnp.dot(g, w2t_ref[...], preferred_element_type=jnp.float32))
    o_ref[...] = jnp.dot(h3, w1t_ref[...], preferred_element_type=jnp.float32)


def _tc3(u2, sparts, W2pT, W1T):
    return pl.pallas_call(
        _tc3_body,
        grid=(NPAD // BN,),
        in_specs=[
            pl.BlockSpec((2, BN, D2), lambda i: (0, i, 0)),
            pl.BlockSpec((32, BN), lambda i: (0, i)),
            pl.BlockSpec((D2, HID), lambda i: (0, 0)),
            pl.BlockSpec((HID, IN_DIM), lambda i: (0, 0)),
        ],
        out_specs=pl.BlockSpec((BN, IN_DIM), lambda i: (i, 0)),
        out_shape=jax.ShapeDtypeStruct((NPAD, IN_DIM), jnp.float32),
    )(u2, sparts, W2pT, W1T)


def _sc_a_body(feat, srcp, dstp, asrc, adst,
               u1, sparts, exv,
               asrc_t, adst_t, s_t, sidx, didx, exb, rows, uacc, sem):
    cid = lax.axis_index("c")
    sid = lax.axis_index("s")
    wid = cid * 16 + sid

    pltpu.sync_copy(asrc, asrc_t)
    pltpu.sync_copy(adst, adst_t)

    zv = jnp.zeros((16,), jnp.float32)

    @pl.loop(0, NPAD // 16)
    def _(i):
        s_t[pl.ds(i * 16, 16)] = zv

    @pl.loop(0, CH)
    def _(i):
        for j in range(D1 // 16):
            rows[i, pl.ds(j * 16, 16)] = zv

    for k in range(RPT // CH):
        pltpu.sync_copy(rows, uacc.at[pl.ds(sid * RPT + k * CH, CH)])
    plsc.subcore_barrier()

    ebase = wid * EPW

    @pl.loop(0, NCH)
    def _(c):
        base = ebase + c * CH
        pltpu.sync_copy(srcp.at[pl.ds(base, CH)], sidx)
        pltpu.sync_copy(dstp.at[pl.ds(base, CH)], didx)
        pltpu.async_copy(feat.at[sidx], rows, sem).wait()
        for j in range(CH // 16):
            sv = sidx[pl.ds(j * 16, 16)]
            dv = didx[pl.ds(j * 16, 16)]
            z = plsc.load_gather(asrc_t, [sv]) + plsc.load_gather(adst_t, [dv])
            sig = 1.0 / (1.0 + jnp.exp(-z))
            ex = jnp.exp(sig)
            exb[pl.ds(j * 16, 16)] = ex
            plsc.addupdate_scatter(s_t, [dv], ex)

        @pl.loop(0, CH)
        def _(i):
            ev = plsc.load_gather(exb, [jnp.zeros((16,), jnp.int32) + i])
            for j in range(D1 // 16):
                rows[i, pl.ds(j * 16, 16)] = rows[i, pl.ds(j * 16, 16)] * ev

        pltpu.sync_copy(rows, uacc.at[didx], add=True)
        pltpu.sync_copy(exb, exv.at[pl.ds(base, CH)])

    pltpu.sync_copy(s_t, sparts.at[wid])
    plsc.subcore_barrier()
    for k in range(RPT // CH):
        r0 = sid * RPT + k * CH
        pltpu.sync_copy(uacc.at[pl.ds(r0, CH)], u1.at[cid, pl.ds(r0, CH)])


_sc_a = functools.partial(
    pl.kernel,
    out_type=(
        jax.ShapeDtypeStruct((2, NPAD, D1), jnp.float32),
        jax.ShapeDtypeStruct((32, NPAD), jnp.float32),
        jax.ShapeDtypeStruct((EP,), jnp.float32),
    ),
    mesh=_sc_mesh,
    compiler_params=pltpu.CompilerParams(
        needs_layout_passes=False, use_tc_tiling_on_sc=False),
    scratch_types=[
        pltpu.VMEM((NPAD,), jnp.float32),
        pltpu.VMEM((NPAD,), jnp.float32),
        pltpu.VMEM((NPAD,), jnp.float32),
        pltpu.VMEM((CH,), jnp.int32),
        pltpu.VMEM((CH,), jnp.int32),
        pltpu.VMEM((CH,), jnp.float32),
        pltpu.VMEM((CH, D1), jnp.float32),
        pltpu.VMEM_SHARED((NPAD, D1), jnp.float32),
        pltpu.SemaphoreType.DMA,
    ],
)(_sc_a_body)


def _sc_b_body(h2p, srcp, dstp, exv,
               u2,
               sidx, didx, exb, rows, uacc, sem):
    cid = lax.axis_index("c")
    sid = lax.axis_index("s")
    wid = cid * 16 + sid

    zv = jnp.zeros((16,), jnp.float32)

    @pl.loop(0, CH)
    def _(i):
        for j in range(D2 // 16):
            rows[i, pl.ds(j * 16, 16)] = zv

    for k in range(RPT // CH):
        pltpu.sync_copy(rows, uacc.at[pl.ds(sid * RPT + k * CH, CH)])
    plsc.subcore_barrier()

    ebase = wid * EPW

    @pl.loop(0, NCH)
    def _(c):
        base = ebase + c * CH
        pltpu.sync_copy(srcp.at[pl.ds(base, CH)], sidx)
        pltpu.sync_copy(dstp.at[pl.ds(base, CH)], didx)
        pltpu.sync_copy(exv.at[pl.ds(base, CH)], exb)
        pltpu.async_copy(h2p.at[sidx], rows, sem).wait()

        @pl.loop(0, CH)
        def _(i):
            ev = plsc.load_gather(exb, [jnp.zeros((16,), jnp.int32) + i])
            for j in range(D2 // 16):
                rows[i, pl.ds(j * 16, 16)] = rows[i, pl.ds(j * 16, 16)] * ev

        pltpu.sync_copy(rows, uacc.at[didx], add=True)

    plsc.subcore_barrier()
    for k in range(RPT // CH):
        r0 = sid * RPT + k * CH
        pltpu.sync_copy(uacc.at[pl.ds(r0, CH)], u2.at[cid, pl.ds(r0, CH)])


_sc_b = functools.partial(
    pl.kernel,
    out_type=jax.ShapeDtypeStruct((2, NPAD, D2), jnp.float32),
    mesh=_sc_mesh,
    compiler_params=pltpu.CompilerParams(
        needs_layout_passes=False, use_tc_tiling_on_sc=False),
    scratch_types=[
        pltpu.VMEM((CH,), jnp.int32),
        pltpu.VMEM((CH,), jnp.int32),
        pltpu.VMEM((CH,), jnp.float32),
        pltpu.VMEM((CH, D2), jnp.float32),
        pltpu.VMEM_SHARED((NPAD, D2), jnp.float32),
        pltpu.SemaphoreType.DMA,
    ],
)(_sc_b_body)


def kernel(features, edge_index, W1, att_src1, att_dst1, W2):
    src = edge_index[0]
    dst = edge_index[1]
    pad_e = EP - E
    srcp = jnp.concatenate([src, jnp.zeros((pad_e,), jnp.int32)])
    dstp = jnp.concatenate(
        [dst, N + (jnp.arange(pad_e, dtype=jnp.int32) % (NPAD - N))])
    att2 = jnp.stack([att_src1, att_dst1], axis=1)
    W2p = jnp.pad(W2, ((0, 0), (0, D2 - OUT)))

    a2 = _tc1(features, W1, att2)
    a2p = jnp.pad(a2, ((0, NPAD - N), (0, 0)))
    asrc = a2p[:, 0]
    adst = a2p[:, 1]

    u1, sparts, exv = _sc_a(features, srcp, dstp, asrc, adst)
    h2p = _tc2(u1, sparts, W1, W2p)
    u2 = _sc_b(h2p, srcp, dstp, exv)
    h4p = _tc3(u2, sparts, W2p.T, W1.T)
    return (h2p[:N, :OUT], h4p[:N])


# speedup vs baseline: 20.9450x; 1.4638x over previous
import functools

import jax
import jax.numpy as jnp
from jax import lax
from jax.experimental import pallas as pl
from jax.experimental.pallas import tpu as pltpu
from jax.experimental.pallas import tpu_sc as plsc

N = 10000
E = 320000
IN_DIM, HID, OUT = 128, 512, 30

NPAD = 10240
EP = 327680
EPW = EP // 32
CH = 128
NCH = EPW // CH
D1 = IN_DIM
DSEG = 64
D2 = 32
RPT = NPAD // 16
BN = 1024

_sc_mesh = plsc.VectorSubcoreMesh(core_axis_name="c", subcore_axis_name="s")


def _tc1_body(x_ref, w1_ref, att2_ref, o_ref):
    av = jnp.dot(w1_ref[...], att2_ref[...], preferred_element_type=jnp.float32)
    o_ref[...] = jnp.dot(x_ref[...], av, preferred_element_type=jnp.float32)


def _tc1(features, W1, att2):
    return pl.pallas_call(
        _tc1_body,
        out_shape=jax.ShapeDtypeStruct((N, 2), jnp.float32),
    )(features, W1, att2)


def _elu(x):
    return jnp.where(x > 0, x, jnp.exp(x) - 1.0)


def _tc2_body(u1_ref, sp_ref, w1_ref, w2_ref, o_ref):
    s = jnp.sum(sp_ref[...], axis=0)
    inv = 1.0 / (s + 1e-16)
    u = jnp.concatenate(
        [u1_ref[0, 0] + u1_ref[1, 0], u1_ref[0, 1] + u1_ref[1, 1]], axis=-1)
    g = u * inv[:, None]
    h1 = _elu(jnp.dot(g, w1_ref[...], preferred_element_type=jnp.float32))
    o_ref[...] = jnp.dot(h1, w2_ref[...], preferred_element_type=jnp.float32)


def _tc2(u1, sparts, W1, W2p):
    return pl.pallas_call(
        _tc2_body,
        grid=(NPAD // BN,),
        in_specs=[
            pl.BlockSpec((2, 2, BN, DSEG), lambda i: (0, 0, i, 0)),
            pl.BlockSpec((32, BN), lambda i: (0, i)),
            pl.BlockSpec((IN_DIM, HID), lambda i: (0, 0)),
            pl.BlockSpec((HID, D2), lambda i: (0, 0)),
        ],
        out_specs=pl.BlockSpec((BN, D2), lambda i: (i, 0)),
        out_shape=jax.ShapeDtypeStruct((NPAD, D2), jnp.float32),
    )(u1, sparts, W1, W2p)


def _tc3_body(u2_ref, sp_ref, w2t_ref, w1t_ref, o_ref):
    s = jnp.sum(sp_ref[...], axis=0)
    inv = 1.0 / (s + 1e-16)
    g = (u2_ref[0] + u2_ref[1]) * inv[:, None]
    h3 = _elu(jnp.dot(g, w2t_ref[...], preferred_element_type=jnp.float32))
    o_ref[...] = jnp.dot(h3, w1t_ref[...], preferred_element_type=jnp.float32)


def _tc3(u2, sparts, W2pT, W1T):
    return pl.pallas_call(
        _tc3_body,
        grid=(NPAD // BN,),
        in_specs=[
            pl.BlockSpec((2, BN, D2), lambda i: (0, i, 0)),
            pl.BlockSpec((32, BN), lambda i: (0, i)),
            pl.BlockSpec((D2, HID), lambda i: (0, 0)),
            pl.BlockSpec((HID, IN_DIM), lambda i: (0, 0)),
        ],
        out_specs=pl.BlockSpec((BN, IN_DIM), lambda i: (i, 0)),
        out_shape=jax.ShapeDtypeStruct((NPAD, IN_DIM), jnp.float32),
    )(u2, sparts, W2pT, W1T)


def _zero_rows(rows, width):
    zv = jnp.zeros((16,), jnp.float32)

    @pl.loop(0, CH)
    def _(i):
        for j in range(width // 16):
            rows[i, pl.ds(j * 16, 16)] = zv


def _sc_a_body(featA, featB, srcp, dstp, asrc, adst,
               u1, sparts, exv,
               asrc_t, adst_t, s_t, src_all, dst_all, ex_all,
               rows0, rows1, uacc, sem0, sem1):
    cid = lax.axis_index("c")
    sid = lax.axis_index("s")
    wid = cid * 16 + sid

    pltpu.sync_copy(asrc, asrc_t)
    pltpu.sync_copy(adst, adst_t)
    pltpu.sync_copy(srcp.at[wid], src_all)
    pltpu.sync_copy(dstp.at[wid], dst_all)

    zv = jnp.zeros((16,), jnp.float32)

    @pl.loop(0, NPAD // 16)
    def _(i):
        s_t[pl.ds(i * 16, 16)] = zv

    @pl.loop(0, NCH)
    def _(c):
        for j in range(CH // 16):
            sv = src_all[c, pl.ds(j * 16, 16)]
            dv = dst_all[c, pl.ds(j * 16, 16)]
            z = plsc.load_gather(asrc_t, [sv]) + plsc.load_gather(adst_t, [dv])
            sig = 1.0 / (1.0 + jnp.exp(-z))
            ex = jnp.exp(sig)
            ex_all[c, pl.ds(j * 16, 16)] = ex
            plsc.addupdate_scatter(s_t, [dv], ex)

    pltpu.sync_copy(ex_all, exv.at[wid])
    pltpu.sync_copy(s_t, sparts.at[wid])

    rows = (rows0, rows1)
    sems = (sem0, sem1)
    for seg, feat in enumerate((featA, featB)):
        _zero_rows(rows0, DSEG)
        for k in range(RPT // CH):
            pltpu.sync_copy(rows0, uacc.at[pl.ds(sid * RPT + k * CH, CH)])
        plsc.subcore_barrier()

        for b in range(2):
            pltpu.async_copy(feat.at[src_all.at[b]], rows[b], sems[b])

        def _chunk(c, b, prefetch, feat=feat):
            pltpu.make_async_copy(feat.at[src_all.at[c]], rows[b], sems[b]).wait()
            zc = jnp.zeros((16,), jnp.int32) + c

            @pl.loop(0, CH, unroll=4)
            def _(i):
                ev = plsc.load_gather(ex_all, [zc, jnp.zeros((16,), jnp.int32) + i])
                for j in range(DSEG // 16):
                    rows[b][i, pl.ds(j * 16, 16)] = rows[b][i, pl.ds(j * 16, 16)] * ev

            pltpu.sync_copy(rows[b], uacc.at[dst_all.at[c]], add=True)
            if prefetch:
                pltpu.async_copy(feat.at[src_all.at[c + 2]], rows[b], sems[b])

        @pl.loop(0, NCH // 2 - 1)
        def _(g):
            for b in range(2):
                _chunk(2 * g + b, b, True)

        for b in range(2):
            _chunk(NCH - 2 + b, b, False)

        plsc.subcore_barrier()
        for k in range(RPT // CH):
            r0 = sid * RPT + k * CH
            pltpu.sync_copy(uacc.at[pl.ds(r0, CH)], u1.at[cid, seg, pl.ds(r0, CH)])
        plsc.subcore_barrier()


_sc_a = functools.partial(
    pl.kernel,
    out_type=(
        jax.ShapeDtypeStruct((2, 2, NPAD, DSEG), jnp.float32),
        jax.ShapeDtypeStruct((32, NPAD), jnp.float32),
        jax.ShapeDtypeStruct((32, NCH, CH), jnp.float32),
    ),
    mesh=_sc_mesh,
    compiler_params=pltpu.CompilerParams(
        needs_layout_passes=False, use_tc_tiling_on_sc=False),
    scratch_types=[
        pltpu.VMEM((NPAD,), jnp.float32),
        pltpu.VMEM((NPAD,), jnp.float32),
        pltpu.VMEM((NPAD,), jnp.float32),
        pltpu.VMEM((NCH, CH), jnp.int32),
        pltpu.VMEM((NCH, CH), jnp.int32),
        pltpu.VMEM((NCH, CH), jnp.float32),
        pltpu.VMEM((CH, DSEG), jnp.float32),
        pltpu.VMEM((CH, DSEG), jnp.float32),
        pltpu.VMEM_SHARED((NPAD, DSEG), jnp.float32),
        pltpu.SemaphoreType.DMA,
        pltpu.SemaphoreType.DMA,
    ],
)(_sc_a_body)


def _sc_b_body(h2p, srcp, dstp, exv,
               u2,
               src_all, dst_all, ex_all, rows0, rows1, uacc, sem0, sem1):
    cid = lax.axis_index("c")
    sid = lax.axis_index("s")
    wid = cid * 16 + sid

    pltpu.sync_copy(srcp.at[wid], src_all)
    pltpu.sync_copy(dstp.at[wid], dst_all)
    pltpu.sync_copy(exv.at[wid], ex_all)

    _zero_rows(rows0, D2)
    for k in range(RPT // CH):
        pltpu.sync_copy(rows0, uacc.at[pl.ds(sid * RPT + k * CH, CH)])
    plsc.subcore_barrier()

    rows = (rows0, rows1)
    sems = (sem0, sem1)
    for b in range(2):
        pltpu.async_copy(h2p.at[src_all.at[b]], rows[b], sems[b])

    def _chunk(c, b, prefetch):
        pltpu.make_async_copy(h2p.at[src_all.at[c]], rows[b], sems[b]).wait()
        zc = jnp.zeros((16,), jnp.int32) + c

        @pl.loop(0, CH, unroll=4)
        def _(i):
            ev = plsc.load_gather(ex_all, [zc, jnp.zeros((16,), jnp.int32) + i])
            for j in range(D2 // 16):
                rows[b][i, pl.ds(j * 16, 16)] = rows[b][i, pl.ds(j * 16, 16)] * ev

        pltpu.sync_copy(rows[b], uacc.at[dst_all.at[c]], add=True)
        if prefetch:
            pltpu.async_copy(h2p.at[src_all.at[c + 2]], rows[b], sems[b])

    @pl.loop(0, NCH // 2 - 1)
    def _(g):
        for b in range(2):
            _chunk(2 * g + b, b, True)

    for b in range(2):
        _chunk(NCH - 2 + b, b, False)

    plsc.subcore_barrier()
    for k in range(RPT // CH):
        r0 = sid * RPT + k * CH
        pltpu.sync_copy(uacc.at[pl.ds(r0, CH)], u2.at[cid, pl.ds(r0, CH)])


_sc_b = functools.partial(
    pl.kernel,
    out_type=jax.ShapeDtypeStruct((2, NPAD, D2), jnp.float32),
    mesh=_sc_mesh,
    compiler_params=pltpu.CompilerParams(
        needs_layout_passes=False, use_tc_tiling_on_sc=False),
    scratch_types=[
        pltpu.VMEM((NCH, CH), jnp.int32),
        pltpu.VMEM((NCH, CH), jnp.int32),
        pltpu.VMEM((NCH, CH), jnp.float32),
        pltpu.VMEM((CH, D2), jnp.float32),
        pltpu.VMEM((CH, D2), jnp.float32),
        pltpu.VMEM_SHARED((NPAD, D2), jnp.float32),
        pltpu.SemaphoreType.DMA,
        pltpu.SemaphoreType.DMA,
    ],
)(_sc_b_body)


def kernel(features, edge_index, W1, att_src1, att_dst1, W2):
    src = edge_index[0]
    dst = edge_index[1]
    pad_e = EP - E
    srcp = jnp.concatenate(
        [src, jnp.zeros((pad_e,), jnp.int32)]).reshape(32, NCH, CH)
    dstp = jnp.concatenate(
        [dst, N + (jnp.arange(pad_e, dtype=jnp.int32) % (NPAD - N))]
    ).reshape(32, NCH, CH)
    att2 = jnp.stack([att_src1, att_dst1], axis=1)
    W2p = jnp.pad(W2, ((0, 0), (0, D2 - OUT)))

    a2 = _tc1(features, W1, att2)
    a2p = jnp.pad(a2, ((0, NPAD - N), (0, 0)))
    asrc = a2p[:, 0]
    adst = a2p[:, 1]

    u1, sparts, exv = _sc_a(
        features[:, :DSEG], features[:, DSEG:], srcp, dstp, asrc, adst)
    h2p = _tc2(u1, sparts, W1, W2p)
    u2 = _sc_b(h2p, srcp, dstp, exv)
    h4p = _tc3(u2, sparts, W2p.T, W1.T)
    return (h2p[:N, :OUT], h4p[:N])


# speedup vs baseline: 21.3116x; 1.0175x over previous
import functools

import jax
import jax.numpy as jnp
from jax import lax
from jax.experimental import pallas as pl
from jax.experimental.pallas import tpu as pltpu
from jax.experimental.pallas import tpu_sc as plsc

N = 10000
E = 320000
IN_DIM, HID, OUT = 128, 512, 30

NPAD = 10240
EP = 327680
EPW = EP // 32
CH = 128
NCH = EPW // CH
D1 = IN_DIM
DSEG = 64
D2 = 32
RPT = NPAD // 16
BN = 1024

_sc_mesh = plsc.VectorSubcoreMesh(core_axis_name="c", subcore_axis_name="s")


def _tc1_body(x_ref, w1_ref, att2_ref, o_ref):
    av = jnp.dot(w1_ref[...], att2_ref[...], preferred_element_type=jnp.float32)
    o_ref[...] = jnp.dot(x_ref[...], av, preferred_element_type=jnp.float32)


def _tc1(features, W1, att2):
    return pl.pallas_call(
        _tc1_body,
        out_shape=jax.ShapeDtypeStruct((N, 2), jnp.float32),
    )(features, W1, att2)


def _elu(x):
    return jnp.where(x > 0, x, jnp.exp(x) - 1.0)


def _tc2_body(u1_ref, sp_ref, w1_ref, w2_ref, o_ref):
    s = jnp.sum(sp_ref[...], axis=0)
    inv = 1.0 / (s + 1e-16)
    u = jnp.concatenate(
        [u1_ref[0, 0] + u1_ref[1, 0], u1_ref[0, 1] + u1_ref[1, 1]], axis=-1)
    g = u * inv[:, None]
    h1 = _elu(jnp.dot(g, w1_ref[...], preferred_element_type=jnp.float32))
    o_ref[...] = jnp.dot(h1, w2_ref[...], preferred_element_type=jnp.float32)


def _tc2(u1, sparts, W1, W2p):
    return pl.pallas_call(
        _tc2_body,
        grid=(NPAD // BN,),
        in_specs=[
            pl.BlockSpec((2, 2, BN, DSEG), lambda i: (0, 0, i, 0)),
            pl.BlockSpec((32, BN), lambda i: (0, i)),
            pl.BlockSpec((IN_DIM, HID), lambda i: (0, 0)),
            pl.BlockSpec((HID, D2), lambda i: (0, 0)),
        ],
        out_specs=pl.BlockSpec((BN, D2), lambda i: (i, 0)),
        out_shape=jax.ShapeDtypeStruct((NPAD, D2), jnp.float32),
    )(u1, sparts, W1, W2p)


def _tc3_body(u2_ref, sp_ref, w2t_ref, w1t_ref, o_ref):
    s = jnp.sum(sp_ref[...], axis=0)
    inv = 1.0 / (s + 1e-16)
    g = (u2_ref[0] + u2_ref[1]) * inv[:, None]
    h3 = _elu(jnp.dot(g, w2t_ref[...], preferred_element_type=jnp.float32))
    o_ref[...] = jnp.dot(h3, w1t_ref[...], preferred_element_type=jnp.float32)


def _tc3(u2, sparts, W2pT, W1T):
    return pl.pallas_call(
        _tc3_body,
        grid=(NPAD // BN,),
        in_specs=[
            pl.BlockSpec((2, BN, D2), lambda i: (0, i, 0)),
            pl.BlockSpec((32, BN), lambda i: (0, i)),
            pl.BlockSpec((D2, HID), lambda i: (0, 0)),
            pl.BlockSpec((HID, IN_DIM), lambda i: (0, 0)),
        ],
        out_specs=pl.BlockSpec((BN, IN_DIM), lambda i: (i, 0)),
        out_shape=jax.ShapeDtypeStruct((NPAD, IN_DIM), jnp.float32),
    )(u2, sparts, W2pT, W1T)


def _zero_rows(rows, width):
    zv = jnp.zeros((16,), jnp.float32)

    @pl.loop(0, CH)
    def _(i):
        for j in range(width // 16):
            rows[i, pl.ds(j * 16, 16)] = zv


NBUF = 4
LEAD = NBUF - 2
DELAY = 2


def _ring_pass(width, feat, src_all, dst_all, ex_all, uacc, rows, gsems, ssems):
    def _iter(m, b, do_wait, do_prefetch):
        if do_wait:
            bp = (b + NBUF - DELAY) % NBUF
            pltpu.make_async_copy(
                rows[bp], uacc.at[dst_all.at[m - DELAY]], ssems[bp]).wait()
        if do_prefetch:
            bl = (b + LEAD) % NBUF
            pltpu.async_copy(feat.at[src_all.at[m + LEAD]], rows[bl], gsems[bl])
        pltpu.make_async_copy(feat.at[src_all.at[m]], rows[b], gsems[b]).wait()
        zc = jnp.zeros((16,), jnp.int32) + m

        @pl.loop(0, CH, unroll=4)
        def _(i):
            ev = plsc.load_gather(ex_all, [zc, jnp.zeros((16,), jnp.int32) + i])
            for j in range(width // 16):
                rows[b][i, pl.ds(j * 16, 16)] = rows[b][i, pl.ds(j * 16, 16)] * ev

        pltpu.async_copy(rows[b], uacc.at[dst_all.at[m]], ssems[b], add=True)

    for m in range(LEAD):
        pltpu.async_copy(feat.at[src_all.at[m]], rows[m % NBUF], gsems[m % NBUF])
    for m in range(NBUF):
        _iter(m, m % NBUF, m >= DELAY, m + LEAD < NCH)

    @pl.loop(1, NCH // NBUF - 1)
    def _(g):
        for b in range(NBUF):
            _iter(NBUF * g + b, b, True, True)

    for m in range(NCH - NBUF, NCH):
        _iter(m, m % NBUF, m + LEAD < NCH, m + LEAD < NCH)
    for m in range(NCH - NBUF, NCH):
        b = m % NBUF
        pltpu.make_async_copy(rows[b], uacc.at[dst_all.at[m]], ssems[b]).wait()


def _sc_a_body(featA, featB, srcp, dstp, asrc, adst,
               u1, sparts, exv,
               s_t, src_all, dst_all, ex_all, uacc,
               gs0, gs1, gs2, gs3, ss0, ss1, ss2, ss3):
    cid = lax.axis_index("c")
    sid = lax.axis_index("s")
    wid = cid * 16 + sid

    pltpu.sync_copy(srcp.at[wid], src_all)
    pltpu.sync_copy(dstp.at[wid], dst_all)

    zv = jnp.zeros((16,), jnp.float32)

    @pl.loop(0, NPAD // 16)
    def _(i):
        s_t[pl.ds(i * 16, 16)] = zv

    def _phase1(asrc_t, adst_t):
        pltpu.sync_copy(asrc, asrc_t)
        pltpu.sync_copy(adst, adst_t)

        @pl.loop(0, NCH)
        def _(c):
            for j in range(CH // 16):
                sv = src_all[c, pl.ds(j * 16, 16)]
                dv = dst_all[c, pl.ds(j * 16, 16)]
                z = plsc.load_gather(asrc_t, [sv]) + plsc.load_gather(adst_t, [dv])
                sig = 1.0 / (1.0 + jnp.exp(-z))
                ex = jnp.exp(sig)
                ex_all[c, pl.ds(j * 16, 16)] = ex
                plsc.addupdate_scatter(s_t, [dv], ex)

    pl.run_scoped(_phase1,
                  pltpu.VMEM((NPAD,), jnp.float32),
                  pltpu.VMEM((NPAD,), jnp.float32))

    pltpu.sync_copy(ex_all, exv.at[wid])
    pltpu.sync_copy(s_t, sparts.at[wid])

    def _phase2(rows0, rows1, rows2, rows3):
        rows = (rows0, rows1, rows2, rows3)
        gsems = (gs0, gs1, gs2, gs3)
        ssems = (ss0, ss1, ss2, ss3)
        for seg, feat in enumerate((featA, featB)):
            _zero_rows(rows0, DSEG)
            for k in range(RPT // CH):
                pltpu.sync_copy(rows0, uacc.at[pl.ds(sid * RPT + k * CH, CH)])
            plsc.subcore_barrier()

            _ring_pass(DSEG, feat, src_all, dst_all, ex_all, uacc,
                       rows, gsems, ssems)

            plsc.subcore_barrier()
            for k in range(RPT // CH):
                r0 = sid * RPT + k * CH
                pltpu.async_copy(uacc.at[pl.ds(r0, CH)],
                                 u1.at[cid, seg, pl.ds(r0, CH)], gs0)
            for k in range(RPT // CH):
                r0 = sid * RPT + k * CH
                pltpu.make_async_copy(uacc.at[pl.ds(r0, CH)],
                                      u1.at[cid, seg, pl.ds(r0, CH)], gs0).wait()
            plsc.subcore_barrier()

    pl.run_scoped(_phase2, *([pltpu.VMEM((CH, DSEG), jnp.float32)] * NBUF))


_sc_a = functools.partial(
    pl.kernel,
    out_type=(
        jax.ShapeDtypeStruct((2, 2, NPAD, DSEG), jnp.float32),
        jax.ShapeDtypeStruct((32, NPAD), jnp.float32),
        jax.ShapeDtypeStruct((32, NCH, CH), jnp.float32),
    ),
    mesh=_sc_mesh,
    compiler_params=pltpu.CompilerParams(
        needs_layout_passes=False, use_tc_tiling_on_sc=False),
    scratch_types=[
        pltpu.VMEM((NPAD,), jnp.float32),
        pltpu.VMEM((NCH, CH), jnp.int32),
        pltpu.VMEM((NCH, CH), jnp.int32),
        pltpu.VMEM((NCH, CH), jnp.float32),
        pltpu.VMEM_SHARED((NPAD, DSEG), jnp.float32),
    ] + [pltpu.SemaphoreType.DMA] * 8,
)(_sc_a_body)


def _sc_b_body(h2p, srcp, dstp, exv,
               u2,
               src_all, dst_all, ex_all,
               rows0, rows1, rows2, rows3, uacc,
               gs0, gs1, gs2, gs3, ss0, ss1, ss2, ss3):
    cid = lax.axis_index("c")
    sid = lax.axis_index("s")
    wid = cid * 16 + sid

    pltpu.sync_copy(srcp.at[wid], src_all)
    pltpu.sync_copy(dstp.at[wid], dst_all)
    pltpu.sync_copy(exv.at[wid], ex_all)

    _zero_rows(rows0, D2)
    for k in range(RPT // CH):
        pltpu.sync_copy(rows0, uacc.at[pl.ds(sid * RPT + k * CH, CH)])
    plsc.subcore_barrier()

    _ring_pass(D2, h2p, src_all, dst_all, ex_all, uacc,
               (rows0, rows1, rows2, rows3),
               (gs0, gs1, gs2, gs3), (ss0, ss1, ss2, ss3))

    plsc.subcore_barrier()
    for k in range(RPT // CH):
        r0 = sid * RPT + k * CH
        pltpu.async_copy(uacc.at[pl.ds(r0, CH)],
                         u2.at[cid, pl.ds(r0, CH)], gs0)
    for k in range(RPT // CH):
        r0 = sid * RPT + k * CH
        pltpu.make_async_copy(uacc.at[pl.ds(r0, CH)],
                              u2.at[cid, pl.ds(r0, CH)], gs0).wait()


_sc_b = functools.partial(
    pl.kernel,
    out_type=jax.ShapeDtypeStruct((2, NPAD, D2), jnp.float32),
    mesh=_sc_mesh,
    compiler_params=pltpu.CompilerParams(
        needs_layout_passes=False, use_tc_tiling_on_sc=False),
    scratch_types=[
        pltpu.VMEM((NCH, CH), jnp.int32),
        pltpu.VMEM((NCH, CH), jnp.int32),
        pltpu.VMEM((NCH, CH), jnp.float32),
        pltpu.VMEM((CH, D2), jnp.float32),
        pltpu.VMEM((CH, D2), jnp.float32),
        pltpu.VMEM((CH, D2), jnp.float32),
        pltpu.VMEM((CH, D2), jnp.float32),
        pltpu.VMEM_SHARED((NPAD, D2), jnp.float32),
    ] + [pltpu.SemaphoreType.DMA] * 8,
)(_sc_b_body)


def kernel(features, edge_index, W1, att_src1, att_dst1, W2):
    src = edge_index[0]
    dst = edge_index[1]
    pad_e = EP - E
    srcp = jnp.concatenate(
        [src, jnp.zeros((pad_e,), jnp.int32)]).reshape(32, NCH, CH)
    dstp = jnp.concatenate(
        [dst, N + (jnp.arange(pad_e, dtype=jnp.int32) % (NPAD - N))]
    ).reshape(32, NCH, CH)
    att2 = jnp.stack([att_src1, att_dst1], axis=1)
    W2p = jnp.pad(W2, ((0, 0), (0, D2 - OUT)))

    a2 = _tc1(features, W1, att2)
    a2p = jnp.pad(a2, ((0, NPAD - N), (0, 0)))
    asrc = a2p[:, 0]
    adst = a2p[:, 1]

    u1, sparts, exv = _sc_a(
        features[:, :DSEG], features[:, DSEG:], srcp, dstp, asrc, adst)
    h2p = _tc2(u1, sparts, W1, W2p)
    u2 = _sc_b(h2p, srcp, dstp, exv)
    h4p = _tc3(u2, sparts, W2p.T, W1.T)
    return (h2p[:N, :OUT], h4p[:N])
